# baseline (speedup 1.0000x reference)
import functools

import jax
import jax.numpy as jnp
from jax import lax
from jax.experimental import pallas as pl
from jax.experimental.pallas import tpu as pltpu

B = 16384
F = 208
NF = 26
E = 16
FE = F * E
BBC = 2048
FEB = 1664


def _builder_body(vt_ref, fi_ref, w_ref, mt_ref, wr_ref, qr_ref):
    fi = fi_ref[...]
    onehot_t = (fi == lax.broadcasted_iota(jnp.int32, (NF, F), 0)).astype(jnp.float32)
    emb_t = jnp.dot(vt_ref[...], onehot_t, preferred_element_type=jnp.float32)

    vbig = pltpu.repeat(emb_t, F, axis=0)
    c_idx = lax.broadcasted_iota(jnp.int32, (FE, F), 0) // E
    f_idx = lax.broadcasted_iota(jnp.int32, (FE, F), 1)
    mt_ref[...] = jnp.where(c_idx == f_idx, vbig, 0.0).astype(jnp.bfloat16)

    s = jnp.sum(emb_t, axis=0, keepdims=True)
    q = jnp.sum(emb_t * emb_t, axis=0, keepdims=True)
    wr_ref[...] = jnp.concatenate([w_ref[...] * (1.0 / F), s * (1.0 / FE)], axis=0)
    qr_ref[...] = q * (1.0 / FE)


def _main_body(xt_ref, mt_ref, wr_ref, qr_ref, out_ref, fm_ref):
    c = pl.program_id(0)
    r = pl.program_id(1)
    xt = xt_ref[...]
    out_ref[...] = jnp.dot(
        mt_ref[pl.ds(r * FEB, FEB), :],
        xt.astype(jnp.bfloat16),
        preferred_element_type=jnp.float32,
    )

    @pl.when(r == 0)
    def _():
        a = jnp.dot(wr_ref[...], xt, preferred_element_type=jnp.float32)
        t2 = jnp.dot(qr_ref[...], xt * xt, preferred_element_type=jnp.float32)
        lin = a[0:1, :]
        t1 = a[1:2, :]
        fm_ref[:, pl.ds(c * BBC, BBC)] = jnp.concatenate(
            [lin, 0.5 * (t1 * t1 - t2)], axis=0
        )


@functools.partial(jax.jit, static_argnames=())
def kernel(inputs, w, V, field_index):
    fi_row = field_index.reshape(1, F).astype(jnp.int32)
    w_row = w.reshape(1, F)
    xt = inputs.T

    mt, wr, qr = pl.pallas_call(
        _builder_body,
        out_shape=(
            jax.ShapeDtypeStruct((FE, F), jnp.bfloat16),
            jax.ShapeDtypeStruct((2, F), jnp.float32),
            jax.ShapeDtypeStruct((1, F), jnp.float32),
        ),
    )(V.T, fi_row, w_row)

    grid = (B // BBC, FE // FEB)
    out_t, fm_t = pl.pallas_call(
        _main_body,
        grid=grid,
        in_specs=[
            pl.BlockSpec((F, BBC), lambda c, r: (0, c)),
            pl.BlockSpec((FE, F), lambda c, r: (0, 0)),
            pl.BlockSpec((2, F), lambda c, r: (0, 0)),
            pl.BlockSpec((1, F), lambda c, r: (0, 0)),
        ],
        out_specs=(
            pl.BlockSpec((FEB, BBC), lambda c, r: (r, c)),
            pl.BlockSpec((2, B), lambda c, r: (0, 0)),
        ),
        out_shape=(
            jax.ShapeDtypeStruct((FE, B), jnp.float32),
            jax.ShapeDtypeStruct((2, B), jnp.float32),
        ),
        compiler_params=pltpu.CompilerParams(
            dimension_semantics=("parallel", "parallel"),
        ),
    )(xt, mt, wr, qr)

    pre = out_t.reshape(F, E, B).transpose(2, 0, 1)
    return fm_t.T, pre


# speedup vs baseline: 1.0371x; 1.0371x over previous
import functools

import jax
import jax.numpy as jnp
from jax import lax
from jax.experimental import pallas as pl
from jax.experimental.pallas import tpu as pltpu

B = 16384
F = 208
NF = 26
E = 16
FE = F * E
BBC = 1024


def _main_body(vt_ref, fi_ref, w_ref, xt_ref, out_ref, fm_ref, mt_ref, wq_ref):
    i = pl.program_id(0)

    @pl.when(i == 0)
    def _():
        fi = fi_ref[...]
        onehot_t = (fi == lax.broadcasted_iota(jnp.int32, (NF, F), 0)).astype(
            jnp.float32
        )
        emb_t = jnp.dot(vt_ref[...], onehot_t, preferred_element_type=jnp.float32)

        vbig = pltpu.repeat(emb_t, F, axis=0)
        c_idx = lax.broadcasted_iota(jnp.int32, (FE, F), 0) // E
        f_idx = lax.broadcasted_iota(jnp.int32, (FE, F), 1)
        mt_ref[...] = jnp.where(c_idx == f_idx, vbig, 0.0).astype(jnp.bfloat16)

        s = jnp.sum(emb_t, axis=0, keepdims=True)
        q = jnp.sum(emb_t * emb_t, axis=0, keepdims=True)
        wq_ref[...] = jnp.concatenate(
            [w_ref[...] * (1.0 / F), s * (1.0 / FE), q * (1.0 / FE)], axis=0
        )

    xt = xt_ref[...]
    out_ref[...] = jnp.dot(
        mt_ref[...], xt.astype(jnp.bfloat16), preferred_element_type=jnp.float32
    )
    a = jnp.dot(wq_ref[0:2, :], xt, preferred_element_type=jnp.float32)
    t2 = jnp.dot(wq_ref[2:3, :], xt * xt, preferred_element_type=jnp.float32)
    lin = a[0:1, :]
    t1 = a[1:2, :]
    fm_ref[:, pl.ds(i * BBC, BBC)] = jnp.concatenate(
        [lin, 0.5 * (t1 * t1 - t2)], axis=0
    )


@functools.partial(jax.jit, static_argnames=())
def kernel(inputs, w, V, field_index):
    fi_row = field_index.reshape(1, F).astype(jnp.int32)
    w_row = w.reshape(1, F)
    xt = inputs.T

    grid = (B // BBC,)
    out_t, fm_t = pl.pallas_call(
        _main_body,
        grid=grid,
        in_specs=[
            pl.BlockSpec((E, NF), lambda i: (0, 0)),
            pl.BlockSpec((1, F), lambda i: (0, 0)),
            pl.BlockSpec((1, F), lambda i: (0, 0)),
            pl.BlockSpec((F, BBC), lambda i: (0, i)),
        ],
        out_specs=(
            pl.BlockSpec((FE, BBC), lambda i: (0, i)),
            pl.BlockSpec((2, B), lambda i: (0, 0)),
        ),
        out_shape=(
            jax.ShapeDtypeStruct((FE, B), jnp.float32),
            jax.ShapeDtypeStruct((2, B), jnp.float32),
        ),
        scratch_shapes=[
            pltpu.VMEM((FE, F), jnp.bfloat16),
            pltpu.VMEM((3, F), jnp.float32),
        ],
        compiler_params=pltpu.CompilerParams(
            dimension_semantics=("arbitrary",),
        ),
    )(V.T, fi_row, w_row, xt)

    pre = out_t.reshape(F, E, B).transpose(2, 0, 1)
    return fm_t.T, pre
